# baseline (speedup 1.0000x reference)
import functools
import jax
import jax.numpy as jnp
from jax.experimental import pallas as pl
from jax.experimental.pallas import tpu as pltpu

N_NODES = 10000
N_GRAPHS = 128


def _matmul_body(x_ref, w_ref, o_ref):
    o_ref[...] = jnp.dot(x_ref[...], w_ref[...],
                         preferred_element_type=jnp.float32)


def _pallas_matmul(x, w):
    M, K = x.shape
    H = w.shape[1]
    Kp = ((K + 127) // 128) * 128
    if Kp != K:
        x = jnp.pad(x, ((0, 0), (0, Kp - K)))
        w = jnp.pad(w, ((0, 0), (0, 0))) if False else jnp.pad(w, ((0, Kp - K), (0, 0)))
    BM = 2000
    grid = (M // BM,)
    return pl.pallas_call(
        _matmul_body,
        grid=grid,
        in_specs=[
            pl.BlockSpec((BM, Kp), lambda i: (i, 0)),
            pl.BlockSpec((Kp, H), lambda i: (0, 0)),
        ],
        out_specs=pl.BlockSpec((BM, H), lambda i: (i, 0)),
        out_shape=jax.ShapeDtypeStruct((M, H), jnp.float32),
    )(x, w)


def _gcn_conv_scaffold(x, W, b, edge_index, edge_weight):
    N = x.shape[0]
    src = edge_index[0]
    dst = edge_index[1]
    loop = jnp.arange(N, dtype=src.dtype)
    src_f = jnp.concatenate([src, loop])
    dst_f = jnp.concatenate([dst, loop])
    ew_f = jnp.concatenate([edge_weight, jnp.ones((N,), dtype=edge_weight.dtype)])
    deg = jnp.zeros((N,), dtype=edge_weight.dtype).at[dst_f].add(ew_f)
    deg_inv_sqrt = jnp.where(deg > 0, 1.0 / jnp.sqrt(deg), 0.0)
    norm = deg_inv_sqrt[src_f] * ew_f * deg_inv_sqrt[dst_f]
    h = _pallas_matmul(x, W)
    msg = h[src_f] * norm[:, None]
    out = jnp.zeros((N, h.shape[1]), dtype=h.dtype).at[dst_f].add(msg)
    return out + b


def kernel(x, edge_index, batch, edge_weight, W1, b1, W2, b2, Wlin, blin):
    h = _gcn_conv_scaffold(x, W1, b1, edge_index, edge_weight)
    h = jax.nn.relu(h)
    h = _gcn_conv_scaffold(h, W2, b2, edge_index, edge_weight)
    sums = jax.ops.segment_sum(h, batch, num_segments=N_GRAPHS)
    counts = jax.ops.segment_sum(jnp.ones((h.shape[0],), dtype=h.dtype), batch,
                                 num_segments=N_GRAPHS)
    pooled = sums / jnp.maximum(counts, 1.0)[:, None]
    return pooled @ Wlin + blin


# speedup vs baseline: 8.3358x; 8.3358x over previous
import functools
import jax
import jax.numpy as jnp
from jax import lax
from jax.experimental import pallas as pl
from jax.experimental.pallas import tpu as pltpu
from jax.experimental.pallas import tpu_sc as plsc

N = 10000
NP = 10240
E = 160000
IN_CH = 268
KP = 384
HID = 512
G = 128

NC = 2
NS = 16
EB = 128
KB = 80
E_PAD = NS * KB * EB
RPT = NP // NS
NCH = 4
CW = 128
CPS = NCH // NC

BM = 2048
NBLK = NP // BM

def _lane_bcast(vec16, lane):
    idx = jnp.full((16,), lane, jnp.int32)
    return vec16.at[idx].get(mode="promise_in_bounds")


_mesh = plsc.VectorSubcoreMesh(core_axis_name="c", subcore_axis_name="s")


DW = 16


@functools.partial(
    pl.kernel,
    mesh=_mesh,
    out_type=jax.ShapeDtypeStruct((NC, NP * DW), jnp.float32),
    scratch_types=[
        pltpu.VMEM((KB * EB,), jnp.int32),
        pltpu.VMEM((KB * EB,), jnp.float32),
        pltpu.VMEM((EB,), jnp.int32),
        pltpu.VMEM((2048,), jnp.float32),
        pltpu.VMEM_SHARED((NP * DW,), jnp.float32),
    ],
)
def _deg_kernel(dst_hbm, ew_hbm, deg_hbm, dst_t, ew_v, idxb, zbuf, acc_sh):
    c = lax.axis_index("c")
    s = lax.axis_index("s")
    pltpu.sync_copy(dst_hbm.at[s], dst_t)
    pltpu.sync_copy(ew_hbm.at[s], ew_v)
    r0 = s * RPT * DW

    def zero16(j, carry):
        zbuf[pl.ds(j * 16, 16)] = jnp.zeros((16,), jnp.float32)
        return carry

    lax.fori_loop(0, 2048 // 16, zero16, 0)
    for j in range(RPT * DW // 2048):
        pltpu.sync_copy(zbuf, acc_sh.at[pl.ds(r0 + j * 2048, 2048)])
    plsc.subcore_barrier()
    k0 = c * (KB // NC)

    def body(i, carry):
        k = k0 + i

        def scale_idx(g, cc):
            d16 = dst_t[pl.ds(k * EB + g * 16, 16)]
            idxb[pl.ds(g * 16, 16)] = d16 * DW
            return cc

        lax.fori_loop(0, EB // 16, scale_idx, 0)
        pltpu.sync_copy(ew_v.at[pl.ds(k * EB, EB)], acc_sh.at[idxb],
                        add=True)
        return carry

    lax.fori_loop(0, KB // NC, body, 0)
    plsc.subcore_barrier()
    pltpu.sync_copy(acc_sh.at[pl.ds(r0, RPT * DW)],
                    deg_hbm.at[c].at[pl.ds(r0, RPT * DW)])


@functools.partial(
    pl.kernel,
    mesh=_mesh,
    out_type=jax.ShapeDtypeStruct((NCH, NP, CW), jnp.float32),
    scratch_types=[
        pltpu.VMEM((KB, EB), jnp.int32),
        pltpu.VMEM((KB, EB), jnp.int32),
        pltpu.VMEM((KB * EB,), jnp.float32),
        pltpu.VMEM((EB, CW), jnp.float32),
        pltpu.VMEM_SHARED((NP, CW), jnp.float32),
        pltpu.SemaphoreType.DMA,
    ],
)
def _agg_kernel(hs_hbm, src_hbm, dst_hbm, ew_hbm, out_hbm,
                src_t, dst_t, ew_v, rows, acc_sh, sem):
    c = lax.axis_index("c")
    s = lax.axis_index("s")
    pltpu.sync_copy(src_hbm.at[s], src_t)
    pltpu.sync_copy(dst_hbm.at[s], dst_t)
    pltpu.sync_copy(ew_hbm.at[s], ew_v)
    r0 = s * RPT
    for ci in range(CPS):
        ch = c * CPS + ci
        pltpu.sync_copy(hs_hbm.at[ch].at[pl.ds(r0, RPT)],
                        acc_sh.at[pl.ds(r0, RPT)])
        plsc.subcore_barrier()

        def body(k, carry):
            pltpu.async_copy(hs_hbm.at[ch].at[src_t.at[k]], rows, sem).wait()
            def egroup(g, cc):
                ew16 = ew_v[pl.ds(k * EB + g * 16, 16)]
                for lane in range(16):
                    w16 = _lane_bcast(ew16, lane)
                    e = g * 16 + lane
                    for v in range(CW // 16):
                        sl = pl.ds(v * 16, 16)
                        rows[e, sl] = rows[e, sl] * w16
                return cc

            lax.fori_loop(0, EB // 16, egroup, 0)
            pltpu.sync_copy(rows, acc_sh.at[dst_t.at[k]], add=True)
            return carry

        lax.fori_loop(0, KB, body, 0)
        plsc.subcore_barrier()
        pltpu.sync_copy(acc_sh.at[pl.ds(r0, RPT)],
                        out_hbm.at[ch].at[pl.ds(r0, RPT)])
        plsc.subcore_barrier()


def _dis_block(deg_ref, i):
    d = (deg_ref[0, pl.ds(i * BM, BM), 0:1]
         + deg_ref[1, pl.ds(i * BM, BM), 0:1])
    return lax.rsqrt(d + 1.0)


def _k2_body(x_ref, w_ref, deg_ref, hs_ref):
    i = pl.program_id(0)
    dis = _dis_block(deg_ref, i)
    h = jnp.dot(x_ref[...], w_ref[...], preferred_element_type=jnp.float32)
    hs = h * dis
    for ch in range(NCH):
        hs_ref[ch, :, :] = hs[:, ch * CW:(ch + 1) * CW]


def _k2(xp, w1p, deg2):
    return pl.pallas_call(
        _k2_body,
        grid=(NBLK,),
        in_specs=[
            pl.BlockSpec((BM, KP), lambda i: (i, 0)),
            pl.BlockSpec((KP, HID), lambda i: (0, 0)),
            pl.BlockSpec((NC, NP, DW), lambda i: (0, 0, 0)),
        ],
        out_specs=pl.BlockSpec((NCH, BM, CW), lambda i: (0, i, 0)),
        out_shape=jax.ShapeDtypeStruct((NCH, NP, CW), jnp.float32),
    )(xp, w1p, deg2)


def _k4_body(agg_ref, deg_ref, b1_ref, w2_ref, hs_ref):
    i = pl.program_id(0)
    dis = _dis_block(deg_ref, i)
    parts = []
    for ch in range(NCH):
        b = b1_ref[0:1, ch * CW:(ch + 1) * CW]
        parts.append(jax.nn.relu(dis * agg_ref[ch, :, :] + b))
    o1 = jnp.concatenate(parts, axis=1)
    h2 = jnp.dot(o1, w2_ref[...], preferred_element_type=jnp.float32)
    for ch in range(NCH):
        hs_ref[ch, :, :] = dis * h2[:, ch * CW:(ch + 1) * CW]


def _k4(agg1, deg2, b1r, w2):
    return pl.pallas_call(
        _k4_body,
        grid=(NBLK,),
        in_specs=[
            pl.BlockSpec((NCH, BM, CW), lambda i: (0, i, 0)),
            pl.BlockSpec((NC, NP, DW), lambda i: (0, 0, 0)),
            pl.BlockSpec((1, HID), lambda i: (0, 0)),
            pl.BlockSpec((HID, HID), lambda i: (0, 0)),
        ],
        out_specs=pl.BlockSpec((NCH, BM, CW), lambda i: (0, i, 0)),
        out_shape=jax.ShapeDtypeStruct((NCH, NP, CW), jnp.float32),
    )(agg1, deg2, b1r, w2)


def _k6_body(agg_ref, deg_ref, b2_ref, batch_ref, wlin_ref, blin_ref,
             out_ref, sums_ref, counts_ref):
    i = pl.program_id(0)

    @pl.when(i == 0)
    def _():
        sums_ref[...] = jnp.zeros_like(sums_ref)
        counts_ref[...] = jnp.zeros_like(counts_ref)

    dis = _dis_block(deg_ref, i)
    parts = []
    for ch in range(NCH):
        b = b2_ref[0:1, ch * CW:(ch + 1) * CW]
        parts.append(dis * agg_ref[ch, :, :] + b)
    h2 = jnp.concatenate(parts, axis=1)

    gids = lax.broadcasted_iota(jnp.int32, (1, G), 1)
    p = (batch_ref[...] == gids).astype(jnp.float32)
    dn = (((0,), (0,)), ((), ()))
    sums_ref[...] += lax.dot_general(p, h2, dn,
                                     preferred_element_type=jnp.float32)
    ones = jnp.ones((BM, 1), jnp.float32)
    counts_ref[...] += lax.dot_general(p, ones, dn,
                                       preferred_element_type=jnp.float32)

    @pl.when(i == NBLK - 1)
    def _():
        pooled = sums_ref[...] / jnp.maximum(counts_ref[...], 1.0)
        out_ref[...] = jnp.dot(pooled, wlin_ref[...],
                               preferred_element_type=jnp.float32) + blin_ref[...]


def _k6(agg2, deg2, b2r, batch2, wlinp, blinp):
    out, _, _ = pl.pallas_call(
        _k6_body,
        grid=(NBLK,),
        in_specs=[
            pl.BlockSpec((NCH, BM, CW), lambda i: (0, i, 0)),
            pl.BlockSpec((NC, NP, DW), lambda i: (0, 0, 0)),
            pl.BlockSpec((1, HID), lambda i: (0, 0)),
            pl.BlockSpec((BM, 1), lambda i: (i, 0)),
            pl.BlockSpec((HID, G), lambda i: (0, 0)),
            pl.BlockSpec((1, G), lambda i: (0, 0)),
        ],
        out_specs=[
            pl.BlockSpec((G, G), lambda i: (0, 0)),
            pl.BlockSpec((G, HID), lambda i: (0, 0)),
            pl.BlockSpec((G, 1), lambda i: (0, 0)),
        ],
        out_shape=[
            jax.ShapeDtypeStruct((G, G), jnp.float32),
            jax.ShapeDtypeStruct((G, HID), jnp.float32),
            jax.ShapeDtypeStruct((G, 1), jnp.float32),
        ],
    )(agg2, deg2, b2r, batch2, wlinp, blinp)
    return out


def kernel(x, edge_index, batch, edge_weight, W1, b1, W2, b2, Wlin, blin):
    x = x.astype(jnp.float32)
    src = edge_index[0].astype(jnp.int32)
    dst = edge_index[1].astype(jnp.int32)
    ew = edge_weight.astype(jnp.float32)

    pad = E_PAD - E
    pad_idx = (jnp.arange(pad, dtype=jnp.int32) * 61) % N
    src_t = jnp.concatenate([src, pad_idx]).reshape(NS, KB, EB)
    dst_t = jnp.concatenate([dst, pad_idx]).reshape(NS, KB, EB)
    ew_t = jnp.concatenate([ew, jnp.zeros((pad,), jnp.float32)]
                           ).reshape(NS, KB * EB)

    deg2 = _deg_kernel(dst_t.reshape(NS, KB * EB), ew_t).reshape(NC, NP, DW)

    xp = jnp.pad(x, ((0, NP - N), (0, KP - IN_CH)))
    w1p = jnp.pad(W1.astype(jnp.float32), ((0, KP - IN_CH), (0, 0)))
    hs1 = _k2(xp, w1p, deg2)

    agg1 = _agg_kernel(hs1, src_t, dst_t, ew_t)

    hs2 = _k4(agg1, deg2, b1.reshape(1, HID).astype(jnp.float32),
              W2.astype(jnp.float32))

    agg2 = _agg_kernel(hs2, src_t, dst_t, ew_t)

    wlinp = jnp.pad(Wlin.astype(jnp.float32), ((0, 0), (0, G - 2)))
    blinp = jnp.pad(blin.reshape(1, 2).astype(jnp.float32),
                    ((0, 0), (0, G - 2)))
    out128 = _k6(agg2, deg2, b2.reshape(1, HID).astype(jnp.float32),
                 jnp.pad(batch.astype(jnp.int32), (0, NP - N),
                          constant_values=G).reshape(NP, 1), wlinp, blinp)
    return out128[:, :2]


# speedup vs baseline: 12.8620x; 1.5430x over previous
import functools
import jax
import jax.numpy as jnp
from jax import lax
from jax.experimental import pallas as pl
from jax.experimental.pallas import tpu as pltpu
from jax.experimental.pallas import tpu_sc as plsc

N = 10000
NP = 10240
E = 160000
IN_CH = 268
KP = 384
HID = 512
G = 128

NC = 2
NS = 16
EB = 128
KB = 80
E_PAD = NS * KB * EB
RPT = NP // NS
NCH = 4
CW = 128
CPS = NCH // NC

BM = 2048
NBLK = NP // BM

def _lane_bcast(vec16, lane):
    idx = jnp.full((16,), lane, jnp.int32)
    return vec16.at[idx].get(mode="promise_in_bounds")


_mesh = plsc.VectorSubcoreMesh(core_axis_name="c", subcore_axis_name="s")


DW = 16


@functools.partial(
    pl.kernel,
    mesh=_mesh,
    out_type=jax.ShapeDtypeStruct((NC, NP * DW), jnp.float32),
    scratch_types=[
        pltpu.VMEM((KB * EB,), jnp.int32),
        pltpu.VMEM((KB * EB,), jnp.float32),
        pltpu.VMEM((EB,), jnp.int32),
        pltpu.VMEM((2048,), jnp.float32),
        pltpu.VMEM_SHARED((NP * DW,), jnp.float32),
    ],
)
def _deg_kernel(dst_hbm, ew_hbm, deg_hbm, dst_t, ew_v, idxb, zbuf, acc_sh):
    c = lax.axis_index("c")
    s = lax.axis_index("s")
    pltpu.sync_copy(dst_hbm.at[s], dst_t)
    pltpu.sync_copy(ew_hbm.at[s], ew_v)
    r0 = s * RPT * DW

    def zero16(j, carry):
        zbuf[pl.ds(j * 16, 16)] = jnp.zeros((16,), jnp.float32)
        return carry

    lax.fori_loop(0, 2048 // 16, zero16, 0)
    for j in range(RPT * DW // 2048):
        pltpu.sync_copy(zbuf, acc_sh.at[pl.ds(r0 + j * 2048, 2048)])
    plsc.subcore_barrier()
    k0 = c * (KB // NC)

    def body(i, carry):
        k = k0 + i

        def scale_idx(g, cc):
            d16 = dst_t[pl.ds(k * EB + g * 16, 16)]
            idxb[pl.ds(g * 16, 16)] = d16 * DW
            return cc

        lax.fori_loop(0, EB // 16, scale_idx, 0)
        pltpu.sync_copy(ew_v.at[pl.ds(k * EB, EB)], acc_sh.at[idxb],
                        add=True)
        return carry

    lax.fori_loop(0, KB // NC, body, 0)
    plsc.subcore_barrier()
    pltpu.sync_copy(acc_sh.at[pl.ds(r0, RPT * DW)],
                    deg_hbm.at[c].at[pl.ds(r0, RPT * DW)])


@functools.partial(
    pl.kernel,
    mesh=_mesh,
    out_type=jax.ShapeDtypeStruct((NCH, NP, CW), jnp.float32),
    scratch_types=[
        pltpu.VMEM((KB * EB,), jnp.int32),
        pltpu.VMEM((EB, CW), jnp.float32),
        pltpu.VMEM((EB, CW), jnp.float32),
        pltpu.VMEM((EB,), jnp.int32),
        pltpu.VMEM((EB,), jnp.int32),
        pltpu.VMEM((EB,), jnp.float32),
        pltpu.VMEM((EB,), jnp.float32),
        pltpu.VMEM_SHARED((NP, CW), jnp.float32),
        pltpu.SemaphoreType.DMA,
        pltpu.SemaphoreType.DMA,
        pltpu.SemaphoreType.DMA,
        pltpu.SemaphoreType.DMA,
    ],
)
def _agg_kernel(hs_hbm, srcf_hbm, dst_hbm, ew_hbm, out_hbm,
                src_v, rows_a, rows_b, dstb_a, dstb_b, ewb_a, ewb_b,
                acc_sh, semr_a, semr_b, semx_a, semx_b):
    c = lax.axis_index("c")
    s = lax.axis_index("s")
    pltpu.sync_copy(srcf_hbm.at[s], src_v)
    r0 = s * RPT

    def chunk_body(ci, chunk_carry):
        ch = c * CPS + ci
        pltpu.sync_copy(hs_hbm.at[ch].at[pl.ds(r0, RPT)],
                        acc_sh.at[pl.ds(r0, RPT)])
        plsc.subcore_barrier()

        def issue(k, rows, dstb, ewb, semr, semx):
            pltpu.async_copy(
                hs_hbm.at[ch].at[src_v.at[pl.ds(k * EB, EB)]], rows, semr)
            pltpu.async_copy(dst_hbm.at[s].at[k], dstb, semx)
            pltpu.async_copy(ew_hbm.at[s].at[k], ewb, semx)

        def drain(rows, dstb, ewb, semr, semx):
            pltpu.make_async_copy(hs_hbm.at[ch].at[pl.ds(0, EB)],
                                  rows, semr).wait()
            pltpu.make_async_copy(dst_hbm.at[s].at[0], dstb, semx).wait()
            pltpu.make_async_copy(ew_hbm.at[s].at[0], ewb, semx).wait()

        def phase(k, rows, dstb, ewb, semr, semx):
            drain(rows, dstb, ewb, semr, semx)

            def egroup(g, cc):
                ew16 = ewb[pl.ds(g * 16, 16)]
                for lane in range(16):
                    w16 = _lane_bcast(ew16, lane)
                    e = g * 16 + lane
                    for v in range(CW // 16):
                        sl = pl.ds(v * 16, 16)
                        rows[e, sl] = rows[e, sl] * w16
                return cc

            lax.fori_loop(0, EB // 16, egroup, 0)
            pltpu.sync_copy(rows, acc_sh.at[dstb], add=True)
            kq = jnp.minimum(k + 2, KB - 1)
            issue(kq, rows, dstb, ewb, semr, semx)

        issue(0, rows_a, dstb_a, ewb_a, semr_a, semx_a)
        issue(1, rows_b, dstb_b, ewb_b, semr_b, semx_b)

        def body(k2, carry):
            k = 2 * k2
            phase(k, rows_a, dstb_a, ewb_a, semr_a, semx_a)
            phase(k + 1, rows_b, dstb_b, ewb_b, semr_b, semx_b)
            return carry

        lax.fori_loop(0, KB // 2, body, 0)
        drain(rows_a, dstb_a, ewb_a, semr_a, semx_a)
        drain(rows_b, dstb_b, ewb_b, semr_b, semx_b)
        plsc.subcore_barrier()
        pltpu.sync_copy(acc_sh.at[pl.ds(r0, RPT)],
                        out_hbm.at[ch].at[pl.ds(r0, RPT)])
        plsc.subcore_barrier()
        return chunk_carry

    lax.fori_loop(0, CPS, chunk_body, 0)


def _dis_block(deg_ref, i):
    d = (deg_ref[0, pl.ds(i * BM, BM), 0:1]
         + deg_ref[1, pl.ds(i * BM, BM), 0:1])
    return lax.rsqrt(d + 1.0)


def _k2_body(x_ref, w_ref, deg_ref, hs_ref):
    i = pl.program_id(0)
    dis = _dis_block(deg_ref, i)
    h = jnp.dot(x_ref[...], w_ref[...], preferred_element_type=jnp.float32)
    hs = h * dis
    for ch in range(NCH):
        hs_ref[ch, :, :] = hs[:, ch * CW:(ch + 1) * CW]


def _k2(xp, w1p, deg2):
    return pl.pallas_call(
        _k2_body,
        grid=(NBLK,),
        in_specs=[
            pl.BlockSpec((BM, KP), lambda i: (i, 0)),
            pl.BlockSpec((KP, HID), lambda i: (0, 0)),
            pl.BlockSpec((NC, NP, DW), lambda i: (0, 0, 0)),
        ],
        out_specs=pl.BlockSpec((NCH, BM, CW), lambda i: (0, i, 0)),
        out_shape=jax.ShapeDtypeStruct((NCH, NP, CW), jnp.float32),
    )(xp, w1p, deg2)


def _k4_body(agg_ref, deg_ref, b1_ref, w2_ref, hs_ref):
    i = pl.program_id(0)
    dis = _dis_block(deg_ref, i)
    parts = []
    for ch in range(NCH):
        b = b1_ref[0:1, ch * CW:(ch + 1) * CW]
        parts.append(jax.nn.relu(dis * agg_ref[ch, :, :] + b))
    o1 = jnp.concatenate(parts, axis=1)
    h2 = jnp.dot(o1, w2_ref[...], preferred_element_type=jnp.float32)
    for ch in range(NCH):
        hs_ref[ch, :, :] = dis * h2[:, ch * CW:(ch + 1) * CW]


def _k4(agg1, deg2, b1r, w2):
    return pl.pallas_call(
        _k4_body,
        grid=(NBLK,),
        in_specs=[
            pl.BlockSpec((NCH, BM, CW), lambda i: (0, i, 0)),
            pl.BlockSpec((NC, NP, DW), lambda i: (0, 0, 0)),
            pl.BlockSpec((1, HID), lambda i: (0, 0)),
            pl.BlockSpec((HID, HID), lambda i: (0, 0)),
        ],
        out_specs=pl.BlockSpec((NCH, BM, CW), lambda i: (0, i, 0)),
        out_shape=jax.ShapeDtypeStruct((NCH, NP, CW), jnp.float32),
    )(agg1, deg2, b1r, w2)


def _k6_body(agg_ref, deg_ref, b2_ref, batch_ref, wlin_ref, blin_ref,
             out_ref, sums_ref, counts_ref):
    i = pl.program_id(0)

    @pl.when(i == 0)
    def _():
        sums_ref[...] = jnp.zeros_like(sums_ref)
        counts_ref[...] = jnp.zeros_like(counts_ref)

    dis = _dis_block(deg_ref, i)
    parts = []
    for ch in range(NCH):
        b = b2_ref[0:1, ch * CW:(ch + 1) * CW]
        parts.append(dis * agg_ref[ch, :, :] + b)
    h2 = jnp.concatenate(parts, axis=1)

    gids = lax.broadcasted_iota(jnp.int32, (1, G), 1)
    p = (batch_ref[...] == gids).astype(jnp.float32)
    dn = (((0,), (0,)), ((), ()))
    sums_ref[...] += lax.dot_general(p, h2, dn,
                                     preferred_element_type=jnp.float32)
    ones = jnp.ones((BM, 1), jnp.float32)
    counts_ref[...] += lax.dot_general(p, ones, dn,
                                       preferred_element_type=jnp.float32)

    @pl.when(i == NBLK - 1)
    def _():
        pooled = sums_ref[...] / jnp.maximum(counts_ref[...], 1.0)
        out_ref[...] = jnp.dot(pooled, wlin_ref[...],
                               preferred_element_type=jnp.float32) + blin_ref[...]


def _k6(agg2, deg2, b2r, batch2, wlinp, blinp):
    out, _, _ = pl.pallas_call(
        _k6_body,
        grid=(NBLK,),
        in_specs=[
            pl.BlockSpec((NCH, BM, CW), lambda i: (0, i, 0)),
            pl.BlockSpec((NC, NP, DW), lambda i: (0, 0, 0)),
            pl.BlockSpec((1, HID), lambda i: (0, 0)),
            pl.BlockSpec((BM, 1), lambda i: (i, 0)),
            pl.BlockSpec((HID, G), lambda i: (0, 0)),
            pl.BlockSpec((1, G), lambda i: (0, 0)),
        ],
        out_specs=[
            pl.BlockSpec((G, G), lambda i: (0, 0)),
            pl.BlockSpec((G, HID), lambda i: (0, 0)),
            pl.BlockSpec((G, 1), lambda i: (0, 0)),
        ],
        out_shape=[
            jax.ShapeDtypeStruct((G, G), jnp.float32),
            jax.ShapeDtypeStruct((G, HID), jnp.float32),
            jax.ShapeDtypeStruct((G, 1), jnp.float32),
        ],
    )(agg2, deg2, b2r, batch2, wlinp, blinp)
    return out


def kernel(x, edge_index, batch, edge_weight, W1, b1, W2, b2, Wlin, blin):
    x = x.astype(jnp.float32)
    src = edge_index[0].astype(jnp.int32)
    dst = edge_index[1].astype(jnp.int32)
    ew = edge_weight.astype(jnp.float32)

    pad = E_PAD - E
    pad_idx = (jnp.arange(pad, dtype=jnp.int32) * 61) % N
    srcf = jnp.concatenate([src, pad_idx]).reshape(NS, KB * EB)
    dst3 = jnp.concatenate([dst, pad_idx]).reshape(NS, KB, EB)
    ew3 = jnp.concatenate([ew, jnp.zeros((pad,), jnp.float32)]
                          ).reshape(NS, KB, EB)

    deg2 = _deg_kernel(dst3.reshape(NS, KB * EB),
                       ew3.reshape(NS, KB * EB)).reshape(NC, NP, DW)

    xp = jnp.pad(x, ((0, NP - N), (0, KP - IN_CH)))
    w1p = jnp.pad(W1.astype(jnp.float32), ((0, KP - IN_CH), (0, 0)))
    hs1 = _k2(xp, w1p, deg2)

    agg1 = _agg_kernel(hs1, srcf, dst3, ew3)

    hs2 = _k4(agg1, deg2, b1.reshape(1, HID).astype(jnp.float32),
              W2.astype(jnp.float32))

    agg2 = _agg_kernel(hs2, srcf, dst3, ew3)

    wlinp = jnp.pad(Wlin.astype(jnp.float32), ((0, 0), (0, G - 2)))
    blinp = jnp.pad(blin.reshape(1, 2).astype(jnp.float32),
                    ((0, 0), (0, G - 2)))
    out128 = _k6(agg2, deg2, b2.reshape(1, HID).astype(jnp.float32),
                 jnp.pad(batch.astype(jnp.int32), (0, NP - N),
                          constant_values=G).reshape(NP, 1), wlinp, blinp)
    return out128[:, :2]
